# baseline (speedup 1.0000x reference)
import functools

import jax
import jax.numpy as jnp
from jax import lax
from jax.experimental import pallas as pl
from jax.experimental.pallas import tpu as pltpu
from jax.experimental.pallas import tpu_sc as plsc

N = 10000
D = 128
WL = 64
WD = 16
NACC = 10016
E = 320000
NC, NS = 2, 16
KC = 80
NT = 250
EPT = NT * KC
NBUF = 4
NGRPF = NT // NBUF - 1
NWIN = 10
WPAIR = EPT // NWIN
ZROWS = NACC // NS


def _sc_body(x2_hbm, ei_hbm, out_hbm, outdeg_hbm,
             src_v, dst_v, rows, ones_v, pairs_v, acc, accdeg, *sems):
    c = lax.axis_index("c")
    s = lax.axis_index("s")
    gsems = sems[:NBUF]
    ssems = sems[NBUF:2 * NBUF]
    dsem = sems[2 * NBUF]

    e0 = jnp.where(lax.iota(jnp.int32, 16) == 0, 1.0, 0.0).astype(jnp.float32)
    def orow(i, carry):
        ones_v[i, pl.ds(0, 16)] = e0
        return carry
    lax.fori_loop(0, KC, orow, 0)

    zb = rows.at[0]
    def zrow(i, carry):
        r = i // (WL // 16)
        col = (i % (WL // 16)) * 16
        zb[r, pl.ds(col, 16)] = jnp.zeros((16,), jnp.float32)
        return carry
    lax.fori_loop(0, KC * WL // 16, zrow, 0)
    row0 = s * ZROWS
    nfull = ZROWS // KC
    for j in range(nfull):
        pltpu.sync_copy(zb, acc.at[pl.ds(row0 + j * KC, KC)])
        pltpu.sync_copy(zb.at[pl.ds(0, KC), pl.ds(0, WD)],
                        accdeg.at[pl.ds(row0 + j * KC, KC)])
    rem = ZROWS - nfull * KC
    if rem:
        pltpu.sync_copy(zb.at[pl.ds(0, rem)], acc.at[pl.ds(row0 + nfull * KC, rem)])
        pltpu.sync_copy(zb.at[pl.ds(0, rem), pl.ds(0, WD)],
                        accdeg.at[pl.ds(row0 + nfull * KC, rem)])
    plsc.subcore_barrier()

    lane = lax.iota(jnp.int32, 16)
    col0 = jnp.zeros((16,), jnp.int32)
    col1 = jnp.ones((16,), jnp.int32)
    for w in range(NWIN):
        pltpu.sync_copy(ei_hbm.at[s, w], pairs_v)
        def deint(i, carry):
            rowidx = i * 16 + lane
            s16 = plsc.load_gather(pairs_v, [rowidx, col0])
            d16 = plsc.load_gather(pairs_v, [rowidx, col1])
            base = w * WPAIR + i * 16
            src_v[pl.ds(base, 16)] = s16 + s16 + c
            dst_v[pl.ds(base, 16)] = d16
            return carry
        lax.fori_loop(0, WPAIR // 16, deint, 0)

    def fire_gather(g, b):
        pltpu.async_copy(
            x2_hbm.at[src_v.at[pl.ds(g * KC, KC)]], rows.at[b], gsems[b])

    def wait_gather(g, b):
        pltpu.make_async_copy(
            x2_hbm.at[src_v.at[pl.ds(g * KC, KC)]], rows.at[b], gsems[b]).wait()

    def fire_scatter(g, b, par):
        pltpu.async_copy(rows.at[b], acc.at[dst_v.at[pl.ds(g * KC, KC)]], ssems[b], add=True)
        @pl.when(c == par)
        def _():
            pltpu.async_copy(ones_v, accdeg.at[dst_v.at[pl.ds(g * KC, KC)]], dsem, add=True)

    def wait_scatter(g, b, par):
        pltpu.make_async_copy(rows.at[b], acc.at[dst_v.at[pl.ds(g * KC, KC)]], ssems[b]).wait()
        @pl.when(c == par)
        def _():
            pltpu.make_async_copy(ones_v, accdeg.at[dst_v.at[pl.ds(g * KC, KC)]], dsem).wait()

    for b in range(NBUF):
        fire_gather(b, b)

    def group(gi, carry):
        for b in range(NBUF):
            g = gi * NBUF + b
            wait_gather(g, b)
            fire_scatter(g, b, b % 2)
            wait_scatter(g, b, b % 2)
            fire_gather(g + NBUF, b)
        return carry
    lax.fori_loop(0, NGRPF, group, 0)

    for g in range(NGRPF * NBUF, NT):
        b = g % NBUF
        wait_gather(g, b)
        fire_scatter(g, b, g % 2)
        wait_scatter(g, b, g % 2)
        if g + NBUF < NT:
            fire_gather(g + NBUF, b)

    plsc.subcore_barrier()

    pltpu.sync_copy(acc.at[pl.ds(row0, ZROWS)],
                    out_hbm.at[pl.ds(row0, ZROWS), pl.ds(c * WL, WL)])
    pltpu.sync_copy(accdeg.at[pl.ds(row0, ZROWS)],
                    outdeg_hbm.at[c, pl.ds(row0, ZROWS)])


@functools.cache
def _sc_scatter():
    return pl.kernel(
        _sc_body,
        out_type=(jax.ShapeDtypeStruct((NACC, D), jnp.float32),
                  jax.ShapeDtypeStruct((NC, NACC, WD), jnp.float32)),
        mesh=plsc.VectorSubcoreMesh(
            core_axis_name="c", subcore_axis_name="s",
            num_cores=NC, num_subcores=NS),
        scratch_types=[
            pltpu.VMEM((EPT,), jnp.int32),
            pltpu.VMEM((EPT,), jnp.int32),
            pltpu.VMEM((NBUF, KC, WL), jnp.float32),
            pltpu.VMEM((KC, WD), jnp.float32),
            pltpu.VMEM((WPAIR, 2), jnp.int32),
            pltpu.VMEM_SHARED((NACC, WL), jnp.float32),
            pltpu.VMEM_SHARED((NACC, WD), jnp.float32),
        ] + [pltpu.SemaphoreType.DMA] * (2 * NBUF + 1),
        compiler_params=pltpu.CompilerParams(use_tc_tiling_on_sc=False, needs_layout_passes=False),
    )


BN = 1000


def _tc_body(x_ref, p_ref, pd_ref, w1a_ref, w1b_ref,
             w2a_ref, w2b_ref, o_ref):
    xb = x_ref[...]
    dg = pd_ref[0, :, 0:1] + pd_ref[1, :, 0:1]
    agg = (jnp.dot(p_ref[...], w1a_ref[...], preferred_element_type=jnp.float32)
           + jnp.dot(xb * dg, w1b_ref[...], preferred_element_type=jnp.float32))
    agg = agg * jnp.float32(1.0 / 9.0)
    o_ref[...] = (
        jnp.dot(jnp.maximum(xb, 0.0), w2a_ref[...], preferred_element_type=jnp.float32)
        + jnp.dot(jnp.maximum(agg, 0.0), w2b_ref[...], preferred_element_type=jnp.float32))


def _tc_finish(x, p, pd, w1a, w1b, w2a, w2b):
    wspec = pl.BlockSpec((D, D), lambda i: (0, 0))
    return pl.pallas_call(
        _tc_body,
        grid=(N // BN,),
        in_specs=[
            pl.BlockSpec((BN, D), lambda i: (i, 0)),
            pl.BlockSpec((BN, D), lambda i: (i, 0)),
            pl.BlockSpec((NC, BN, WD), lambda i: (0, i, 0)),
            wspec, wspec, wspec, wspec,
        ],
        out_specs=pl.BlockSpec((BN, D), lambda i: (i, 0)),
        out_shape=jax.ShapeDtypeStruct((N, D), jnp.float32),
    )(x, p, pd, w1a, w1b, w2a, w2b)


def kernel(x, edge_index, W1, W2):
    ei4 = edge_index.astype(jnp.int32).reshape(NS, NWIN, WPAIR, 2)
    x2 = x.reshape(2 * N, WL)
    p, pd = _sc_scatter()(x2, ei4)
    return _tc_finish(x, p, pd, W1[:D], W1[D:], W2[:D], W2[D:])


# speedup vs baseline: 2.7566x; 2.7566x over previous
import functools

import jax
import jax.numpy as jnp
from jax import lax
from jax.experimental import pallas as pl
from jax.experimental.pallas import tpu as pltpu
from jax.experimental.pallas import tpu_sc as plsc

N = 10000
D = 128
WL = 64
WD = 16
NACC = 10016
E = 320000
NC, NS = 2, 16
KC = 80
NT = 250
EPT = NT * KC
NBUF = 6
NGRPF = NT // NBUF - 1
ZROWS = NACC // NS


def _sc_body(x2_hbm, src_hbm, dst_hbm, out_hbm, outdeg_hbm,
             src_v, dst_v, rows, ones_v, acc, accdeg, *sems):
    c = lax.axis_index("c")
    s = lax.axis_index("s")
    gsems = sems[:NBUF]
    ssems = sems[NBUF:2 * NBUF]
    dsem = sems[2 * NBUF]

    e0 = jnp.where(lax.iota(jnp.int32, 16) == 0, 1.0, 0.0).astype(jnp.float32)
    def orow(i, carry):
        ones_v[i, pl.ds(0, 16)] = e0
        return carry
    lax.fori_loop(0, KC, orow, 0)

    zb = rows.at[0]
    def zrow(i, carry):
        r = i // (WL // 16)
        col = (i % (WL // 16)) * 16
        zb[r, pl.ds(col, 16)] = jnp.zeros((16,), jnp.float32)
        return carry
    lax.fori_loop(0, KC * WL // 16, zrow, 0)
    row0 = s * ZROWS
    nfull = ZROWS // KC
    for j in range(nfull):
        pltpu.sync_copy(zb, acc.at[pl.ds(row0 + j * KC, KC)])
        pltpu.sync_copy(zb.at[pl.ds(0, KC), pl.ds(0, WD)],
                        accdeg.at[pl.ds(row0 + j * KC, KC)])
    rem = ZROWS - nfull * KC
    if rem:
        pltpu.sync_copy(zb.at[pl.ds(0, rem)], acc.at[pl.ds(row0 + nfull * KC, rem)])
        pltpu.sync_copy(zb.at[pl.ds(0, rem), pl.ds(0, WD)],
                        accdeg.at[pl.ds(row0 + nfull * KC, rem)])
    plsc.subcore_barrier()

    pltpu.sync_copy(src_hbm.at[s], src_v)
    pltpu.sync_copy(dst_hbm.at[s], dst_v)
    def remap(i, carry):
        v = src_v[pl.ds(i * 16, 16)]
        src_v[pl.ds(i * 16, 16)] = v + v + c
        return carry
    lax.fori_loop(0, EPT // 16, remap, 0)

    def fire_gather(g, b):
        pltpu.async_copy(
            x2_hbm.at[src_v.at[pl.ds(g * KC, KC)]], rows.at[b], gsems[b])

    def wait_gather(g, b):
        pltpu.make_async_copy(
            x2_hbm.at[src_v.at[pl.ds(g * KC, KC)]], rows.at[b], gsems[b]).wait()

    def fire_scatter(g, b, par):
        pltpu.async_copy(rows.at[b], acc.at[dst_v.at[pl.ds(g * KC, KC)]], ssems[b], add=True)
        @pl.when(c == par)
        def _():
            pltpu.async_copy(ones_v, accdeg.at[dst_v.at[pl.ds(g * KC, KC)]], dsem, add=True)

    def wait_scatter(g, b, par):
        pltpu.make_async_copy(rows.at[b], acc.at[dst_v.at[pl.ds(g * KC, KC)]], ssems[b]).wait()
        @pl.when(c == par)
        def _():
            pltpu.make_async_copy(ones_v, accdeg.at[dst_v.at[pl.ds(g * KC, KC)]], dsem).wait()

    for b in range(NBUF):
        fire_gather(b, b)

    def group(gi, carry):
        for b in range(NBUF):
            g = gi * NBUF + b
            wait_gather(g, b)
            fire_scatter(g, b, b % 2)
            wait_scatter(g, b, b % 2)
            fire_gather(g + NBUF, b)
        return carry
    lax.fori_loop(0, NGRPF, group, 0)

    for g in range(NGRPF * NBUF, NT):
        b = g % NBUF
        wait_gather(g, b)
        fire_scatter(g, b, g % 2)
        wait_scatter(g, b, g % 2)
        if g + NBUF < NT:
            fire_gather(g + NBUF, b)

    plsc.subcore_barrier()

    pltpu.sync_copy(acc.at[pl.ds(row0, ZROWS)],
                    out_hbm.at[pl.ds(row0, ZROWS), pl.ds(c * WL, WL)])
    pltpu.sync_copy(accdeg.at[pl.ds(row0, ZROWS)],
                    outdeg_hbm.at[c, pl.ds(row0, ZROWS)])


@functools.cache
def _sc_scatter():
    return pl.kernel(
        _sc_body,
        out_type=(jax.ShapeDtypeStruct((NACC, D), jnp.float32),
                  jax.ShapeDtypeStruct((NC, NACC, WD), jnp.float32)),
        mesh=plsc.VectorSubcoreMesh(
            core_axis_name="c", subcore_axis_name="s",
            num_cores=NC, num_subcores=NS),
        scratch_types=[
            pltpu.VMEM((EPT,), jnp.int32),
            pltpu.VMEM((EPT,), jnp.int32),
            pltpu.VMEM((NBUF, KC, WL), jnp.float32),
            pltpu.VMEM((KC, WD), jnp.float32),
            pltpu.VMEM_SHARED((NACC, WL), jnp.float32),
            pltpu.VMEM_SHARED((NACC, WD), jnp.float32),
        ] + [pltpu.SemaphoreType.DMA] * (2 * NBUF + 1),
        compiler_params=pltpu.CompilerParams(use_tc_tiling_on_sc=False),
    )


BN = 1000


def _tc_body(x_ref, p_ref, pd_ref, w1a_ref, w1b_ref,
             w2a_ref, w2b_ref, o_ref):
    xb = x_ref[...]
    dg = pd_ref[0, :, 0:1] + pd_ref[1, :, 0:1]
    agg = (jnp.dot(p_ref[...], w1a_ref[...], preferred_element_type=jnp.float32)
           + jnp.dot(xb * dg, w1b_ref[...], preferred_element_type=jnp.float32))
    agg = agg * jnp.float32(1.0 / 9.0)
    o_ref[...] = (
        jnp.dot(jnp.maximum(xb, 0.0), w2a_ref[...], preferred_element_type=jnp.float32)
        + jnp.dot(jnp.maximum(agg, 0.0), w2b_ref[...], preferred_element_type=jnp.float32))


def _tc_finish(x, p, pd, w1a, w1b, w2a, w2b):
    wspec = pl.BlockSpec((D, D), lambda i: (0, 0))
    return pl.pallas_call(
        _tc_body,
        grid=(N // BN,),
        in_specs=[
            pl.BlockSpec((BN, D), lambda i: (i, 0)),
            pl.BlockSpec((BN, D), lambda i: (i, 0)),
            pl.BlockSpec((NC, BN, WD), lambda i: (0, i, 0)),
            wspec, wspec, wspec, wspec,
        ],
        out_specs=pl.BlockSpec((BN, D), lambda i: (i, 0)),
        out_shape=jax.ShapeDtypeStruct((N, D), jnp.float32),
    )(x, p, pd, w1a, w1b, w2a, w2b)


def kernel(x, edge_index, W1, W2):
    ei = edge_index.astype(jnp.int32).T
    src_p = ei[0].reshape(NS, EPT)
    dst_p = ei[1].reshape(NS, EPT)
    x2 = x.reshape(2 * N, WL)
    p, pd = _sc_scatter()(x2, src_p, dst_p)
    return _tc_finish(x, p, pd, W1[:D], W1[D:], W2[:D], W2[D:])
